# Initial kernel scaffold; baseline (speedup 1.0000x reference)
import jax
import jax.numpy as jnp
from jax.experimental import pallas as pl


def kernel(x, edge_index, W1, b1, W2, b2, W3, b3):
    raise NotImplementedError("write your pallas kernel here")


# speedup vs baseline: 8.8865x; 8.8865x over previous
import functools

import jax
import jax.numpy as jnp
from jax import lax
from jax.experimental import pallas as pl
from jax.experimental.pallas import tpu as pltpu
from jax.experimental.pallas import tpu_sc as plsc

N_REAL = 50000
N_PAD = 50176
E_REAL = 1600000
E_PAD = 1638400
N_CORES = 2
N_SUBCORES = 16
E_PER_CORE = E_PAD // N_CORES
E_PER_TILE = E_PER_CORE // N_SUBCORES
BATCH = 128
N_BATCHES = E_PER_TILE // BATCH
STRIPE = N_PAD // N_SUBCORES
ROWS_BLK = 1024
GRID_ROWS = N_PAD // ROWS_BLK


def _make_prop(fc):
    mesh = plsc.VectorSubcoreMesh(core_axis_name="c", subcore_axis_name="s")

    @functools.partial(
        pl.kernel,
        out_type=jax.ShapeDtypeStruct((N_CORES, N_PAD, fc), jnp.float32),
        mesh=mesh,
        scratch_types=[
            pltpu.VMEM((BATCH,), jnp.int32),
            pltpu.VMEM((BATCH,), jnp.int32),
            pltpu.VMEM((BATCH, fc), jnp.float32),
            pltpu.VMEM_SHARED((N_PAD, fc), jnp.float32),
            pltpu.SemaphoreType.DMA,
        ],
        compiler_params=pltpu.CompilerParams(use_tc_tiling_on_sc=False),
    )
    def prop(table, srcs, dsts, out, sbuf, dbuf, rows, acc, sem):
        c = lax.axis_index("c")
        s = lax.axis_index("s")
        r0 = s * STRIPE
        pltpu.sync_copy(table.at[pl.ds(r0, STRIPE)], acc.at[pl.ds(r0, STRIPE)])
        plsc.subcore_barrier()
        base = c * E_PER_CORE + s * E_PER_TILE

        def body(i, carry):
            off = base + i * BATCH
            pltpu.sync_copy(srcs.at[pl.ds(off, BATCH)], sbuf)
            pltpu.sync_copy(dsts.at[pl.ds(off, BATCH)], dbuf)
            pltpu.async_copy(table.at[sbuf], rows, sem).wait()
            pltpu.sync_copy(rows, acc.at[dbuf], add=True)
            return carry

        lax.fori_loop(0, N_BATCHES, body, 0)
        plsc.subcore_barrier()
        pltpu.sync_copy(acc.at[pl.ds(r0, STRIPE)], out.at[c, pl.ds(r0, STRIPE)])

    return prop


_prop32 = _make_prop(32)
_prop16 = _make_prop(16)
_prop1 = _make_prop(1)


def _stage1_body(degp, xpad, dinv_o, xs32_o, xs16_o):
    deg = degp[0] + degp[1] - 1.0
    row = (pl.program_id(0) * ROWS_BLK
           + lax.broadcasted_iota(jnp.int32, (ROWS_BLK, 1), 0))
    dinv = jnp.where(row < N_REAL, lax.rsqrt(deg), 0.0)
    dinv_o[...] = dinv
    xs = dinv * xpad[...]
    xs32_o[...] = xs[:, :32]
    xs16_o[...] = xs[:, 32:]


def _stage1(degp, xpad):
    blk = lambda *shape: pl.BlockSpec(shape, lambda i: (0,) * (len(shape) - 2) + (i, 0))
    return pl.pallas_call(
        _stage1_body,
        grid=(GRID_ROWS,),
        in_specs=[blk(2, ROWS_BLK, 1), blk(ROWS_BLK, 48)],
        out_specs=(blk(ROWS_BLK, 1), blk(ROWS_BLK, 32), blk(ROWS_BLK, 16)),
        out_shape=(
            jax.ShapeDtypeStruct((N_PAD, 1), jnp.float32),
            jax.ShapeDtypeStruct((N_PAD, 32), jnp.float32),
            jax.ShapeDtypeStruct((N_PAD, 16), jnp.float32),
        ),
    )(degp, xpad)


def _stage2_body(z1a, z1b, xs32, xs16, dinv, w1, b1, w2p,
                 t0_o, t1_o, t2_o, t3_o):
    d = dinv[...]
    za = d * (z1a[0] + z1a[1] - xs32[...])
    zb = d * (z1b[0] + z1b[1] - xs16[...])
    h1 = jnp.dot(za, w1[:32, :], preferred_element_type=jnp.float32)
    h1 = h1 + jnp.dot(zb, w1[32:, :], preferred_element_type=jnp.float32)
    h1 = jnp.maximum(h1 + b1[...], 0.0)
    t2 = d * jnp.dot(h1, w2p[...], preferred_element_type=jnp.float32)
    t0_o[...] = t2[:, 0:32]
    t1_o[...] = t2[:, 32:64]
    t2_o[...] = t2[:, 64:96]
    t3_o[...] = t2[:, 96:112]


def _stage2(z1a, z1b, xs32, xs16, dinv, w1, b1r, w2p):
    blk = lambda *shape: pl.BlockSpec(shape, lambda i: (0,) * (len(shape) - 2) + (i, 0))
    full = lambda *shape: pl.BlockSpec(shape, lambda i: (0,) * len(shape))
    return pl.pallas_call(
        _stage2_body,
        grid=(GRID_ROWS,),
        in_specs=[
            blk(2, ROWS_BLK, 32), blk(2, ROWS_BLK, 16),
            blk(ROWS_BLK, 32), blk(ROWS_BLK, 16), blk(ROWS_BLK, 1),
            full(48, 300), full(1, 300), full(300, 112),
        ],
        out_specs=(
            blk(ROWS_BLK, 32), blk(ROWS_BLK, 32),
            blk(ROWS_BLK, 32), blk(ROWS_BLK, 16),
        ),
        out_shape=tuple(
            jax.ShapeDtypeStruct((N_PAD, w), jnp.float32) for w in (32, 32, 32, 16)
        ),
    )(z1a, z1b, xs32, xs16, dinv, w1, b1r, w2p)


def _stage3_body(za, zb, zc, zd, ta, tb, tc_, td, dinv, b2p, w3p, t3_o):
    d = dinv[...]
    acc = jnp.zeros((za.shape[1], 1), jnp.float32)
    for i, (z, t, lo, hi) in enumerate((
            (za, ta, 0, 32), (zb, tb, 32, 64), (zc, tc_, 64, 96), (zd, td, 96, 112))):
        h = jnp.maximum(d * (z[0] + z[1] - t[...]) + b2p[:, lo:hi], 0.0)
        acc = acc + jnp.dot(h, w3p[lo:hi, :], preferred_element_type=jnp.float32)
    t3_o[...] = d * acc


def _stage3(z2s, t2s, dinv, b2p, w3p):
    blk = lambda *shape: pl.BlockSpec(shape, lambda i: (0,) * (len(shape) - 2) + (i, 0))
    full = lambda *shape: pl.BlockSpec(shape, lambda i: (0,) * len(shape))
    return pl.pallas_call(
        _stage3_body,
        grid=(GRID_ROWS,),
        in_specs=[
            blk(2, ROWS_BLK, 32), blk(2, ROWS_BLK, 32),
            blk(2, ROWS_BLK, 32), blk(2, ROWS_BLK, 16),
            blk(ROWS_BLK, 32), blk(ROWS_BLK, 32),
            blk(ROWS_BLK, 32), blk(ROWS_BLK, 16),
            blk(ROWS_BLK, 1), full(1, 112), full(112, 1),
        ],
        out_specs=blk(ROWS_BLK, 1),
        out_shape=jax.ShapeDtypeStruct((N_PAD, 1), jnp.float32),
    )(*z2s, *t2s, dinv, b2p, w3p)


def _stage4_body(z3, t3, dinv, b3, out_o):
    out_o[...] = dinv[...] * (z3[0] + z3[1] - t3[...]) + b3[0, 0]


def _stage4(z3, t3, dinv, b3):
    blk = lambda *shape: pl.BlockSpec(shape, lambda i: (0,) * (len(shape) - 2) + (i, 0))
    full = lambda *shape: pl.BlockSpec(shape, lambda i: (0,) * len(shape))
    return pl.pallas_call(
        _stage4_body,
        grid=(GRID_ROWS,),
        in_specs=[blk(2, ROWS_BLK, 1), blk(ROWS_BLK, 1), blk(ROWS_BLK, 1),
                  full(1, 1)],
        out_specs=blk(ROWS_BLK, 1),
        out_shape=jax.ShapeDtypeStruct((N_PAD, 1), jnp.float32),
    )(z3, t3, dinv, b3)


def kernel(x, edge_index, W1, b1, W2, b2, W3, b3):
    pad = jnp.full((E_PAD - E_REAL,), N_REAL, dtype=jnp.int32)
    srcs = jnp.concatenate([edge_index[0], pad])
    dsts = jnp.concatenate([edge_index[1], pad])
    xpad = jnp.pad(x, ((0, N_PAD - N_REAL), (0, 0)))
    ones = jnp.ones((N_PAD, 1), jnp.float32)
    b1r = b1.reshape(1, 300)
    w2p = jnp.pad(W2, ((0, 0), (0, 12)))
    b2p = jnp.pad(b2, (0, 12)).reshape(1, 112)
    w3p = jnp.pad(W3, ((0, 12), (0, 0)))
    b3r = b3.reshape(1, 1)

    degp = _prop1(ones, srcs, dsts)
    dinv, xs32, xs16 = _stage1(degp, xpad)
    z1a = _prop32(xs32, srcs, dsts)
    z1b = _prop16(xs16, srcs, dsts)
    t2s = _stage2(z1a, z1b, xs32, xs16, dinv, W1, b1r, w2p)
    z2s = tuple(p(t, srcs, dsts) for p, t in
                zip((_prop32, _prop32, _prop32, _prop16), t2s))
    t3 = _stage3(z2s, t2s, dinv, b2p, w3p)
    z3 = _prop1(t3, srcs, dsts)
    out = _stage4(z3, t3, dinv, b3r)
    return out[:N_REAL, 0]


# speedup vs baseline: 17.5306x; 1.9727x over previous
import functools

import jax
import jax.numpy as jnp
from jax import lax
from jax.experimental import pallas as pl
from jax.experimental.pallas import tpu as pltpu
from jax.experimental.pallas import tpu_sc as plsc

N_REAL = 50000
N_PAD = 50176
E_REAL = 1600000
E_PAD = 1638400
N_CORES = 2
N_SUBCORES = 16
E_PER_CORE = E_PAD // N_CORES
E_PER_TILE = E_PER_CORE // N_SUBCORES
BATCH = 128
N_BATCHES = E_PER_TILE // BATCH
NBUF = 3
CHUNK = 50
STRIPE = N_PAD // N_SUBCORES
ROWS_BLK = 1024
GRID_ROWS = N_PAD // ROWS_BLK


def _make_prop(fc):
    mesh = plsc.VectorSubcoreMesh(core_axis_name="c", subcore_axis_name="s")

    @functools.partial(
        pl.kernel,
        out_type=jax.ShapeDtypeStruct((N_CORES, N_PAD, fc), jnp.float32),
        mesh=mesh,
        scratch_types=[
            pltpu.VMEM((CHUNK, BATCH), jnp.int32),
            pltpu.VMEM((CHUNK, BATCH), jnp.int32),
            [pltpu.VMEM((BATCH, fc), jnp.float32) for _ in range(NBUF)],
            pltpu.VMEM_SHARED((N_PAD, fc), jnp.float32),
            [pltpu.SemaphoreType.DMA for _ in range(NBUF)],
        ],
        compiler_params=pltpu.CompilerParams(use_tc_tiling_on_sc=False),
    )
    def prop(table, srcs2, dsts2, out, sbuf, dbuf, rows, acc, sems):
        c = lax.axis_index("c")
        s = lax.axis_index("s")
        r0 = s * STRIPE
        pltpu.sync_copy(table.at[pl.ds(r0, STRIPE)], acc.at[pl.ds(r0, STRIPE)])
        plsc.subcore_barrier()
        rb0 = c * (E_PER_CORE // BATCH) + s * N_BATCHES

        def chunk_body(k, carry):
            rb = rb0 + k * CHUNK
            pltpu.sync_copy(srcs2.at[pl.ds(rb, CHUNK)], sbuf)
            pltpu.sync_copy(dsts2.at[pl.ds(rb, CHUNK)], dbuf)
            for b in range(NBUF):
                pltpu.async_copy(table.at[sbuf.at[b]], rows[b], sems[b])
            for j in range(CHUNK):
                b = j % NBUF
                pltpu.make_async_copy(table.at[sbuf.at[j]], rows[b], sems[b]).wait()
                pltpu.sync_copy(rows[b], acc.at[dbuf.at[j]], add=True)
                if j + NBUF < CHUNK:
                    pltpu.async_copy(table.at[sbuf.at[j + NBUF]], rows[b], sems[b])
            return carry

        lax.fori_loop(0, N_BATCHES // CHUNK, chunk_body, 0)
        plsc.subcore_barrier()
        pltpu.sync_copy(acc.at[pl.ds(r0, STRIPE)], out.at[c, pl.ds(r0, STRIPE)])

    return prop


_prop32 = _make_prop(32)
_prop16 = _make_prop(16)
_prop1 = _make_prop(1)


def _stage1_body(degp, xpad, dinv_o, xs32_o, xs16_o):
    deg = degp[0] + degp[1] - 1.0
    row = (pl.program_id(0) * ROWS_BLK
           + lax.broadcasted_iota(jnp.int32, (ROWS_BLK, 1), 0))
    dinv = jnp.where(row < N_REAL, lax.rsqrt(deg), 0.0)
    dinv_o[...] = dinv
    xs = dinv * xpad[...]
    xs32_o[...] = xs[:, :32]
    xs16_o[...] = xs[:, 32:]


def _stage1(degp, xpad):
    blk = lambda *shape: pl.BlockSpec(shape, lambda i: (0,) * (len(shape) - 2) + (i, 0))
    return pl.pallas_call(
        _stage1_body,
        grid=(GRID_ROWS,),
        in_specs=[blk(2, ROWS_BLK, 1), blk(ROWS_BLK, 48)],
        out_specs=(blk(ROWS_BLK, 1), blk(ROWS_BLK, 32), blk(ROWS_BLK, 16)),
        out_shape=(
            jax.ShapeDtypeStruct((N_PAD, 1), jnp.float32),
            jax.ShapeDtypeStruct((N_PAD, 32), jnp.float32),
            jax.ShapeDtypeStruct((N_PAD, 16), jnp.float32),
        ),
    )(degp, xpad)


def _stage2_body(z1a, z1b, xs32, xs16, dinv, w1, b1, w2p,
                 t0_o, t1_o, t2_o, t3_o):
    d = dinv[...]
    za = d * (z1a[0] + z1a[1] - xs32[...])
    zb = d * (z1b[0] + z1b[1] - xs16[...])
    h1 = jnp.dot(za, w1[:32, :], preferred_element_type=jnp.float32)
    h1 = h1 + jnp.dot(zb, w1[32:, :], preferred_element_type=jnp.float32)
    h1 = jnp.maximum(h1 + b1[...], 0.0)
    t2 = d * jnp.dot(h1, w2p[...], preferred_element_type=jnp.float32)
    t0_o[...] = t2[:, 0:32]
    t1_o[...] = t2[:, 32:64]
    t2_o[...] = t2[:, 64:96]
    t3_o[...] = t2[:, 96:112]


def _stage2(z1a, z1b, xs32, xs16, dinv, w1, b1r, w2p):
    blk = lambda *shape: pl.BlockSpec(shape, lambda i: (0,) * (len(shape) - 2) + (i, 0))
    full = lambda *shape: pl.BlockSpec(shape, lambda i: (0,) * len(shape))
    return pl.pallas_call(
        _stage2_body,
        grid=(GRID_ROWS,),
        in_specs=[
            blk(2, ROWS_BLK, 32), blk(2, ROWS_BLK, 16),
            blk(ROWS_BLK, 32), blk(ROWS_BLK, 16), blk(ROWS_BLK, 1),
            full(48, 300), full(1, 300), full(300, 112),
        ],
        out_specs=(
            blk(ROWS_BLK, 32), blk(ROWS_BLK, 32),
            blk(ROWS_BLK, 32), blk(ROWS_BLK, 16),
        ),
        out_shape=tuple(
            jax.ShapeDtypeStruct((N_PAD, w), jnp.float32) for w in (32, 32, 32, 16)
        ),
    )(z1a, z1b, xs32, xs16, dinv, w1, b1r, w2p)


def _stage3_body(za, zb, zc, zd, ta, tb, tc_, td, dinv, b2p, w3p, t3_o):
    d = dinv[...]
    acc = jnp.zeros((za.shape[1], 1), jnp.float32)
    for i, (z, t, lo, hi) in enumerate((
            (za, ta, 0, 32), (zb, tb, 32, 64), (zc, tc_, 64, 96), (zd, td, 96, 112))):
        h = jnp.maximum(d * (z[0] + z[1] - t[...]) + b2p[:, lo:hi], 0.0)
        acc = acc + jnp.dot(h, w3p[lo:hi, :], preferred_element_type=jnp.float32)
    t3_o[...] = d * acc


def _stage3(z2s, t2s, dinv, b2p, w3p):
    blk = lambda *shape: pl.BlockSpec(shape, lambda i: (0,) * (len(shape) - 2) + (i, 0))
    full = lambda *shape: pl.BlockSpec(shape, lambda i: (0,) * len(shape))
    return pl.pallas_call(
        _stage3_body,
        grid=(GRID_ROWS,),
        in_specs=[
            blk(2, ROWS_BLK, 32), blk(2, ROWS_BLK, 32),
            blk(2, ROWS_BLK, 32), blk(2, ROWS_BLK, 16),
            blk(ROWS_BLK, 32), blk(ROWS_BLK, 32),
            blk(ROWS_BLK, 32), blk(ROWS_BLK, 16),
            blk(ROWS_BLK, 1), full(1, 112), full(112, 1),
        ],
        out_specs=blk(ROWS_BLK, 1),
        out_shape=jax.ShapeDtypeStruct((N_PAD, 1), jnp.float32),
    )(*z2s, *t2s, dinv, b2p, w3p)


def _stage4_body(z3, t3, dinv, b3, out_o):
    out_o[...] = dinv[...] * (z3[0] + z3[1] - t3[...]) + b3[0, 0]


def _stage4(z3, t3, dinv, b3):
    blk = lambda *shape: pl.BlockSpec(shape, lambda i: (0,) * (len(shape) - 2) + (i, 0))
    full = lambda *shape: pl.BlockSpec(shape, lambda i: (0,) * len(shape))
    return pl.pallas_call(
        _stage4_body,
        grid=(GRID_ROWS,),
        in_specs=[blk(2, ROWS_BLK, 1), blk(ROWS_BLK, 1), blk(ROWS_BLK, 1),
                  full(1, 1)],
        out_specs=blk(ROWS_BLK, 1),
        out_shape=jax.ShapeDtypeStruct((N_PAD, 1), jnp.float32),
    )(z3, t3, dinv, b3)


def kernel(x, edge_index, W1, b1, W2, b2, W3, b3):
    pad = jnp.full((E_PAD - E_REAL,), N_REAL, dtype=jnp.int32)
    srcs = jnp.concatenate([edge_index[0], pad]).reshape(E_PAD // BATCH, BATCH)
    dsts = jnp.concatenate([edge_index[1], pad]).reshape(E_PAD // BATCH, BATCH)
    xpad = jnp.pad(x, ((0, N_PAD - N_REAL), (0, 0)))
    ones = jnp.ones((N_PAD, 1), jnp.float32)
    b1r = b1.reshape(1, 300)
    w2p = jnp.pad(W2, ((0, 0), (0, 12)))
    b2p = jnp.pad(b2, (0, 12)).reshape(1, 112)
    w3p = jnp.pad(W3, ((0, 12), (0, 0)))
    b3r = b3.reshape(1, 1)

    degp = _prop1(ones, srcs, dsts)
    dinv, xs32, xs16 = _stage1(degp, xpad)
    z1a = _prop32(xs32, srcs, dsts)
    z1b = _prop16(xs16, srcs, dsts)
    t2s = _stage2(z1a, z1b, xs32, xs16, dinv, W1, b1r, w2p)
    z2s = tuple(p(t, srcs, dsts) for p, t in
                zip((_prop32, _prop32, _prop32, _prop16), t2s))
    t3 = _stage3(z2s, t2s, dinv, b2p, w3p)
    z3 = _prop1(t3, srcs, dsts)
    out = _stage4(z3, t3, dinv, b3r)
    return out[:N_REAL, 0]
